# baseline (speedup 1.0000x reference)
import functools

import jax
import jax.numpy as jnp
from jax import lax
from jax.experimental import pallas as pl
from jax.experimental.pallas import tpu as pltpu
from jax.experimental.pallas import tpu_sc as plsc

IN_C = 256
HID = 256
OUT_C = 349
OUT_P = 384
N_TOTAL = 10000
N1 = 8000
N2 = 6000
E0 = 160000
E1 = 96000

NC, NS = 2, 16
NW = NC * NS
CW = 100
R0 = E0 // CW
R1 = E1 // CW
HP_ROWS = 8192


def _zero16():
    return jnp.zeros((16,), jnp.float32)


def _rsqrt16(x):
    a = jnp.full_like(x, 1.0)
    t = x
    for _ in range(9):
        big = t >= 4.0
        a = jnp.where(big, a * 2.0, a)
        t = jnp.where(big, t * 0.25, t)
    y = (a / x)
    h = x * jnp.float32(0.5)
    for _ in range(6):
        y = y * (jnp.float32(1.5) - h * y * y)
    return y


def _sc1_body(tbl, lidx8, nt8, e0r, e1r,
              hp0, hp1, d0d, d1s, d1d,
              hA, hB, hC,
              ebuf, ebuf2, ebuf3, ones, zbuf, dall, rbuf, cidx, lbuf, nbuf,
              rows, sem, sem2, sem3, sem4):
    c = lax.axis_index("c")
    s = lax.axis_index("s")
    wid = s * NC + c

    for k in range(32):
        zbuf[pl.ds(k * 16, 16)] = _zero16()
    for k in range(7):
        ones[pl.ds(k * 16, 16)] = jnp.ones((16,), jnp.float32)

    pltpu.sync_copy(zbuf, hA.at[pl.ds(s * 512, 512)])
    pltpu.sync_copy(zbuf, hB.at[pl.ds(s * 512, 512)])
    pltpu.sync_copy(zbuf, hC.at[pl.ds(s * 512, 512)])
    plsc.subcore_barrier()

    def _hists(specs):
        for plane, n, _h, eb, _s in specs:
            pltpu.sync_copy(plane.at[s], eb.at[pl.ds(0, n)])
        nmax = max(n for _p, n, _h, _e, _s in specs)

        def step(j, carry):
            for _p, n, h, eb, sm in specs:
                @pl.when(j < n)
                def _():
                    pltpu.async_copy(ones.at[pl.ds(0, CW)], h.at[eb.at[j]],
                                     sm, add=True)
            for _p, n, h, eb, sm in specs:
                @pl.when(j < n)
                def _():
                    pltpu.make_async_copy(ones.at[pl.ds(0, CW)],
                                          h.at[eb.at[j]], sm).wait()
            return carry

        lax.fori_loop(0, nmax, step, 0)

    @pl.when(c == 0)
    def _():
        _hists([(e0r.at[0], 100, hA, ebuf, sem2),
                (e0r.at[1], 100, hB, ebuf2, sem3)])

    @pl.when(c == 1)
    def _():
        _hists([(e0r.at[0], 100, hA, ebuf, sem2),
                (e1r.at[0], 60, hB, ebuf2, sem3),
                (e1r.at[1], 60, hC, ebuf3, sem4)])

    plsc.subcore_barrier()

    @pl.when((c == 0) & (s == 0))
    def _():
        pltpu.sync_copy(hB.at[pl.ds(0, N1)], dall.at[pl.ds(0, N1)])
        pltpu.sync_copy(dall.at[pl.ds(0, N1)], d0d)

    @pl.when((c == 1) & (s == 0))
    def _():
        pltpu.sync_copy(hB.at[pl.ds(0, N2)], dall.at[pl.ds(0, N2)])
        pltpu.sync_copy(dall.at[pl.ds(0, N2)], d1s)

    @pl.when((c == 1) & (s == 1))
    def _():
        pltpu.sync_copy(hC.at[pl.ds(0, N2)], dall.at[pl.ds(0, N2)])
        pltpu.sync_copy(dall.at[pl.ds(0, N2)], d1d)

    pltpu.sync_copy(hA, dall)
    base = wid * 256

    def rstep(k, carry):
        x = jnp.maximum(dall[pl.ds(base + k * 16, 16)], jnp.float32(1.0))
        rbuf[pl.ds(k * 16, 16)] = _rsqrt16(x)
        return carry

    lax.fori_loop(0, 16, rstep, 0)

    pltpu.sync_copy(lidx8.at[pl.ds(base, 256)], lbuf)
    pltpu.sync_copy(nt8.at[pl.ds(base, 256)], nbuf)

    def cstep(k, carry):
        cidx[pl.ds(k * 16, 16)] = (lbuf[pl.ds(k * 16, 16)]
                                   + nbuf[pl.ds(k * 16, 16)] * N_TOTAL)
        return carry

    lax.fori_loop(0, 16, cstep, 0)

    for g in range(4):
        pltpu.async_copy(tbl.at[cidx.at[pl.ds(g * 64, 64)]], rows, sem).wait()

        def sstep(row, carry, g=g):
            r = rbuf[pl.ds(g * 64 + row, 16)][0]
            for cc in range(16):
                rows[row, pl.ds(cc * 16, 16)] = (
                    rows[row, pl.ds(cc * 16, 16)] * r)
            return carry

        lax.fori_loop(0, 64, sstep, 0)
        pltpu.sync_copy(rows.at[pl.ds(0, 64), pl.ds(0, 128)],
                        hp0.at[pl.ds(base + g * 64, 64)])
        pltpu.sync_copy(rows.at[pl.ds(0, 64), pl.ds(128, 128)],
                        hp1.at[pl.ds(base + g * 64, 64)])


def _sc1(tbl, lidx8, nt8, e0r, e1r):
    mesh = plsc.VectorSubcoreMesh(core_axis_name="c", subcore_axis_name="s",
                                  num_cores=NC, num_subcores=NS)
    f32 = jnp.float32
    out_type = (
        jax.ShapeDtypeStruct((HP_ROWS, IN_C // 2), f32),
        jax.ShapeDtypeStruct((HP_ROWS, IN_C // 2), f32),
        jax.ShapeDtypeStruct((N1,), f32),
        jax.ShapeDtypeStruct((N2,), f32),
        jax.ShapeDtypeStruct((N2,), f32),
    )
    scratch = [
        pltpu.VMEM_SHARED((HP_ROWS,), f32),
        pltpu.VMEM_SHARED((HP_ROWS,), f32),
        pltpu.VMEM_SHARED((HP_ROWS,), f32),
        pltpu.VMEM((100, CW), jnp.int32),
        pltpu.VMEM((100, CW), jnp.int32),
        pltpu.VMEM((60, CW), jnp.int32),
        pltpu.VMEM((112,), f32),
        pltpu.VMEM((512,), f32),
        pltpu.VMEM((HP_ROWS,), f32),
        pltpu.VMEM((272,), f32),
        pltpu.VMEM((256,), jnp.int32),
        pltpu.VMEM((256,), jnp.int32),
        pltpu.VMEM((256,), jnp.int32),
        pltpu.VMEM((64, IN_C), f32),
        pltpu.SemaphoreType.DMA,
        pltpu.SemaphoreType.DMA,
        pltpu.SemaphoreType.DMA,
        pltpu.SemaphoreType.DMA,
    ]
    fn = pl.kernel(_sc1_body, out_type=out_type, mesh=mesh,
                   scratch_types=scratch)
    return fn(tbl, lidx8, nt8, e0r, e1r)


HC = IN_C // 2


def _agg_body(n_acc_rows, cpt, feat0, feat1, er, out, acc, sbuf, dbuf,
              gbuf, semg, sems):
    c = lax.axis_index("c")
    s = lax.axis_index("s")

    def zstep(i, carry):
        row = lax.shift_right_logical(i, 3)
        cc = lax.rem(i, 8)
        gbuf[0, row, pl.ds(cc * 16, 16)] = _zero16()
        return carry

    lax.fori_loop(0, CW * (HC // 16), zstep, 0)

    big = -(-n_acc_rows // NS) // 8 * 8
    last = n_acc_rows - 15 * big

    def _zero_rows(row0, nrows):
        done = 0
        while done < nrows:
            take = min(96, nrows - done)
            pltpu.sync_copy(gbuf.at[0, pl.ds(0, take)],
                            acc.at[pl.ds(row0 + done, take)])
            done += take

    @pl.when(s < 15)
    def _():
        _zero_rows(s * big, big)

    @pl.when(s == 15)
    def _():
        _zero_rows(15 * big, last)

    plsc.subcore_barrier()

    pltpu.sync_copy(er.at[0, s], sbuf)
    pltpu.sync_copy(er.at[1, s], dbuf)

    def _run(feat):
        pltpu.async_copy(feat.at[sbuf.at[0]], gbuf.at[0], semg)

        def step(j, carry):
            b = lax.rem(j, 2)
            pltpu.make_async_copy(feat.at[sbuf.at[j]], gbuf.at[b],
                                  semg).wait()

            @pl.when(j >= 1)
            def _():
                pltpu.make_async_copy(gbuf.at[1 - b],
                                      acc.at[dbuf.at[j - 1]], sems).wait()

            pltpu.async_copy(gbuf.at[b], acc.at[dbuf.at[j]], sems, add=True)

            @pl.when(j + 1 < cpt)
            def _():
                pltpu.async_copy(feat.at[sbuf.at[j + 1]],
                                 gbuf.at[1 - b], semg)

            return carry

        lax.fori_loop(0, cpt, step, 0)
        pltpu.make_async_copy(gbuf.at[(cpt - 1) % 2],
                              acc.at[dbuf.at[cpt - 1]], sems).wait()

    @pl.when(c == 0)
    def _():
        _run(feat0)

    @pl.when(c == 1)
    def _():
        _run(feat1)

    plsc.subcore_barrier()

    def _copy_out(row0, nrows):
        done = 0
        for ch in (96, 96, 96, 88, 72):
            take = min(ch, nrows - done)
            if take <= 0:
                break
            pltpu.sync_copy(acc.at[pl.ds(row0 + done, take)],
                            gbuf.at[0, pl.ds(0, take)])
            pltpu.sync_copy(gbuf.at[0, pl.ds(0, take)],
                            out.at[c, pl.ds(row0 + done, take)])
            done += take

    @pl.when(s < 15)
    def _():
        _copy_out(s * 376, 376)

    @pl.when(s == 15)
    def _():
        _copy_out(15 * 376, 360)


def _edge_agg(feat0, feat1, er, n_acc_rows, cpt):
    mesh = plsc.VectorSubcoreMesh(core_axis_name="c", subcore_axis_name="s",
                                  num_cores=NC, num_subcores=NS)
    f32 = jnp.float32
    scratch = [
        pltpu.VMEM_SHARED((n_acc_rows, HC), f32),
        pltpu.VMEM((cpt, CW), jnp.int32),
        pltpu.VMEM((cpt, CW), jnp.int32),
        pltpu.VMEM((2, CW, HC), f32),
        pltpu.SemaphoreType.DMA,
        pltpu.SemaphoreType.DMA,
    ]
    fn = pl.kernel(functools.partial(_agg_body, n_acc_rows, cpt),
                   out_type=jax.ShapeDtypeStruct((NC, N2, HC), f32),
                   mesh=mesh, scratch_types=scratch)
    return fn(feat0, feat1, er)


_BM = 600


def _tca_body(agg_ref, d0d_ref, d1s_ref, w_ref, b_ref, out0_ref, out1_ref):
    blk = agg_ref[...]
    a = jnp.concatenate((blk[0], blk[1]), axis=-1)
    s1 = lax.rsqrt(jnp.maximum(d0d_ref[0, 0], 1.0))
    s2 = lax.rsqrt(jnp.maximum(d1s_ref[0, 0], 1.0))
    h = jnp.dot(a * s1[:, None], w_ref[...],
                preferred_element_type=jnp.float32) + b_ref[...][None, :]
    x = jnp.maximum(h, 0.0) * s2[:, None]
    out0_ref[...] = x[:, :HC]
    out1_ref[...] = x[:, HC:]


def _tca(agg0, d0d, d1s, W1, b1):
    grid = (N2 // _BM,)
    f32 = jnp.float32
    return pl.pallas_call(
        _tca_body,
        grid=grid,
        in_specs=[
            pl.BlockSpec((NC, _BM, HC), lambda i: (0, i, 0)),
            pl.BlockSpec((1, 1, _BM), lambda i: (i, 0, 0)),
            pl.BlockSpec((1, 1, _BM), lambda i: (i, 0, 0)),
            pl.BlockSpec((IN_C, HID), lambda i: (0, 0)),
            pl.BlockSpec((HID,), lambda i: (0,)),
        ],
        out_specs=[
            pl.BlockSpec((_BM, HC), lambda i: (i, 0)),
            pl.BlockSpec((_BM, HC), lambda i: (i, 0)),
        ],
        out_shape=(
            jax.ShapeDtypeStruct((N2, HC), f32),
            jax.ShapeDtypeStruct((N2, HC), f32),
        ),
    )(agg0, d0d, d1s, W1, b1)


def _tcb_body(agg_ref, d1d_ref, w_ref, b_ref, out_ref):
    blk = agg_ref[...]
    a = jnp.concatenate((blk[0], blk[1]), axis=-1)
    s1 = lax.rsqrt(jnp.maximum(d1d_ref[0, 0], 1.0))
    z = jnp.dot(a * s1[:, None], w_ref[...],
                preferred_element_type=jnp.float32) + b_ref[...][None, :]
    m = jnp.max(z, axis=1, keepdims=True)
    lse = jnp.log(jnp.sum(jnp.exp(z - m), axis=1, keepdims=True)) + m
    out_ref[...] = (z - lse)[:, :OUT_C]


def _tcb(agg1, d1d, W2p, b2p):
    grid = (N2 // _BM,)
    return pl.pallas_call(
        _tcb_body,
        grid=grid,
        in_specs=[
            pl.BlockSpec((NC, _BM, HC), lambda i: (0, i, 0)),
            pl.BlockSpec((1, 1, _BM), lambda i: (i, 0, 0)),
            pl.BlockSpec((HID, OUT_P), lambda i: (0, 0)),
            pl.BlockSpec((OUT_P,), lambda i: (0,)),
        ],
        out_specs=pl.BlockSpec((_BM, OUT_C), lambda i: (i, 0)),
        out_shape=jax.ShapeDtypeStruct((N2, OUT_C), jnp.float32),
    )(agg1, d1d, W2p, b2p)


def kernel(n_id, x0, edge_index0, edge_index1, node_type, local_node_idx,
           emb1, W1, b1, W2, b2):
    tbl = jnp.concatenate([x0, emb1], axis=0)
    lidx8 = local_node_idx[:HP_ROWS]
    nt8 = node_type[:HP_ROWS]
    e0r = edge_index0.reshape(2, NS, E0 // NS // CW, CW)
    e1r = edge_index1.reshape(2, NS, E1 // NS // CW, CW)

    hp0, hp1, d0d, d1s, d1d = _sc1(tbl, lidx8, nt8, e0r, e1r)
    d0d = d0d[:N2].reshape(N2 // _BM, 1, _BM)
    d1s = d1s.reshape(N2 // _BM, 1, _BM)
    d1d = d1d.reshape(N2 // _BM, 1, _BM)
    agg0 = _edge_agg(hp0, hp1, e0r, n_acc_rows=N1, cpt=E0 // NS // CW)
    xp0, xp1 = _tca(agg0, d0d, d1s, W1, b1)
    agg1 = _edge_agg(xp0, xp1, e1r, n_acc_rows=N2, cpt=E1 // NS // CW)

    W2p = jnp.pad(W2, ((0, 0), (0, OUT_P - OUT_C)))
    b2p = jnp.pad(b2, (0, OUT_P - OUT_C), constant_values=-1e30)
    return _tcb(agg1, d1d, W2p, b2p)


# speedup vs baseline: 1.0530x; 1.0530x over previous
import functools

import jax
import jax.numpy as jnp
from jax import lax
from jax.experimental import pallas as pl
from jax.experimental.pallas import tpu as pltpu
from jax.experimental.pallas import tpu_sc as plsc

IN_C = 256
HID = 256
OUT_C = 349
OUT_P = 384
N_TOTAL = 10000
N1 = 8000
N2 = 6000
E0 = 160000
E1 = 96000

NC, NS = 2, 16
NW = NC * NS
CW = 125
R0 = E0 // CW
R1 = E1 // CW
HP_ROWS = 8192


def _zero16():
    return jnp.zeros((16,), jnp.float32)


def _rsqrt16(x):
    a = jnp.full_like(x, 1.0)
    t = x
    for _ in range(9):
        big = t >= 4.0
        a = jnp.where(big, a * 2.0, a)
        t = jnp.where(big, t * 0.25, t)
    y = (a / x)
    h = x * jnp.float32(0.5)
    for _ in range(6):
        y = y * (jnp.float32(1.5) - h * y * y)
    return y


def _sc1_body(tbl, lidx8, nt8, e0r, e1r,
              hp0, hp1, d0d, d1s, d1d,
              hA, hB, hC,
              ebuf, ebuf2, ebuf3, ones, zbuf, dall, rbuf, cidx, lbuf, nbuf,
              rows, sem, sem2, sem3, sem4):
    c = lax.axis_index("c")
    s = lax.axis_index("s")
    wid = s * NC + c

    for k in range(32):
        zbuf[pl.ds(k * 16, 16)] = _zero16()
    for k in range(8):
        ones[pl.ds(k * 16, 16)] = jnp.ones((16,), jnp.float32)

    pltpu.sync_copy(zbuf, hA.at[pl.ds(s * 512, 512)])
    pltpu.sync_copy(zbuf, hB.at[pl.ds(s * 512, 512)])
    pltpu.sync_copy(zbuf, hC.at[pl.ds(s * 512, 512)])
    plsc.subcore_barrier()

    def _hists(specs):
        for plane, n, _h, eb, _s in specs:
            pltpu.sync_copy(plane.at[s], eb.at[pl.ds(0, n)])
        nmax = max(n for _p, n, _h, _e, _s in specs)

        def step(j, carry):
            for _p, n, h, eb, sm in specs:
                @pl.when(j < n)
                def _():
                    pltpu.async_copy(ones.at[pl.ds(0, CW)], h.at[eb.at[j]],
                                     sm, add=True)
            for _p, n, h, eb, sm in specs:
                @pl.when(j < n)
                def _():
                    pltpu.make_async_copy(ones.at[pl.ds(0, CW)],
                                          h.at[eb.at[j]], sm).wait()
            return carry

        lax.fori_loop(0, nmax, step, 0)

    @pl.when(c == 0)
    def _():
        _hists([(e0r.at[0], 80, hA, ebuf, sem2),
                (e0r.at[1], 80, hB, ebuf2, sem3)])

    @pl.when(c == 1)
    def _():
        _hists([(e0r.at[0], 80, hA, ebuf, sem2),
                (e1r.at[0], 48, hB, ebuf2, sem3),
                (e1r.at[1], 48, hC, ebuf3, sem4)])

    plsc.subcore_barrier()

    @pl.when((c == 0) & (s == 0))
    def _():
        pltpu.sync_copy(hB.at[pl.ds(0, N1)], dall.at[pl.ds(0, N1)])
        pltpu.sync_copy(dall.at[pl.ds(0, N1)], d0d)

    @pl.when((c == 1) & (s == 0))
    def _():
        pltpu.sync_copy(hB.at[pl.ds(0, N2)], dall.at[pl.ds(0, N2)])
        pltpu.sync_copy(dall.at[pl.ds(0, N2)], d1s)

    @pl.when((c == 1) & (s == 1))
    def _():
        pltpu.sync_copy(hC.at[pl.ds(0, N2)], dall.at[pl.ds(0, N2)])
        pltpu.sync_copy(dall.at[pl.ds(0, N2)], d1d)

    pltpu.sync_copy(hA, dall)
    base = wid * 256

    def rstep(k, carry):
        x = jnp.maximum(dall[pl.ds(base + k * 16, 16)], jnp.float32(1.0))
        rbuf[pl.ds(k * 16, 16)] = _rsqrt16(x)
        return carry

    lax.fori_loop(0, 16, rstep, 0)

    pltpu.sync_copy(lidx8.at[pl.ds(base, 256)], lbuf)
    pltpu.sync_copy(nt8.at[pl.ds(base, 256)], nbuf)

    def cstep(k, carry):
        cidx[pl.ds(k * 16, 16)] = (lbuf[pl.ds(k * 16, 16)]
                                   + nbuf[pl.ds(k * 16, 16)] * N_TOTAL)
        return carry

    lax.fori_loop(0, 16, cstep, 0)

    for g in range(4):
        pltpu.async_copy(tbl.at[cidx.at[pl.ds(g * 64, 64)]], rows, sem).wait()

        def sstep(row, carry, g=g):
            r = rbuf[pl.ds(g * 64 + row, 16)][0]
            for cc in range(16):
                rows[row, pl.ds(cc * 16, 16)] = (
                    rows[row, pl.ds(cc * 16, 16)] * r)
            return carry

        lax.fori_loop(0, 64, sstep, 0)
        pltpu.sync_copy(rows.at[pl.ds(0, 64), pl.ds(0, 128)],
                        hp0.at[pl.ds(base + g * 64, 64)])
        pltpu.sync_copy(rows.at[pl.ds(0, 64), pl.ds(128, 128)],
                        hp1.at[pl.ds(base + g * 64, 64)])


def _sc1(tbl, lidx8, nt8, e0r, e1r):
    mesh = plsc.VectorSubcoreMesh(core_axis_name="c", subcore_axis_name="s",
                                  num_cores=NC, num_subcores=NS)
    f32 = jnp.float32
    out_type = (
        jax.ShapeDtypeStruct((HP_ROWS, IN_C // 2), f32),
        jax.ShapeDtypeStruct((HP_ROWS, IN_C // 2), f32),
        jax.ShapeDtypeStruct((N1,), f32),
        jax.ShapeDtypeStruct((N2,), f32),
        jax.ShapeDtypeStruct((N2,), f32),
    )
    scratch = [
        pltpu.VMEM_SHARED((HP_ROWS,), f32),
        pltpu.VMEM_SHARED((HP_ROWS,), f32),
        pltpu.VMEM_SHARED((HP_ROWS,), f32),
        pltpu.VMEM((80, CW), jnp.int32),
        pltpu.VMEM((80, CW), jnp.int32),
        pltpu.VMEM((48, CW), jnp.int32),
        pltpu.VMEM((128,), f32),
        pltpu.VMEM((512,), f32),
        pltpu.VMEM((HP_ROWS,), f32),
        pltpu.VMEM((272,), f32),
        pltpu.VMEM((256,), jnp.int32),
        pltpu.VMEM((256,), jnp.int32),
        pltpu.VMEM((256,), jnp.int32),
        pltpu.VMEM((64, IN_C), f32),
        pltpu.SemaphoreType.DMA,
        pltpu.SemaphoreType.DMA,
        pltpu.SemaphoreType.DMA,
        pltpu.SemaphoreType.DMA,
    ]
    fn = pl.kernel(_sc1_body, out_type=out_type, mesh=mesh,
                   scratch_types=scratch)
    return fn(tbl, lidx8, nt8, e0r, e1r)


HC = IN_C // 2


def _agg_body(n_acc_rows, cpt, feat0, feat1, er, out, acc, sbuf, dbuf,
              gbuf, semg, sems):
    c = lax.axis_index("c")
    s = lax.axis_index("s")

    def zstep(i, carry):
        row = lax.shift_right_logical(i, 3)
        cc = lax.rem(i, 8)
        gbuf[0, row, pl.ds(cc * 16, 16)] = _zero16()
        return carry

    lax.fori_loop(0, CW * (HC // 16), zstep, 0)

    big = -(-n_acc_rows // NS) // 8 * 8
    last = n_acc_rows - 15 * big

    def _zero_rows(row0, nrows):
        done = 0
        while done < nrows:
            take = min(96, nrows - done)
            pltpu.sync_copy(gbuf.at[0, pl.ds(0, take)],
                            acc.at[pl.ds(row0 + done, take)])
            done += take

    @pl.when(s < 15)
    def _():
        _zero_rows(s * big, big)

    @pl.when(s == 15)
    def _():
        _zero_rows(15 * big, last)

    plsc.subcore_barrier()

    pltpu.sync_copy(er.at[0, s], sbuf)
    pltpu.sync_copy(er.at[1, s], dbuf)

    def _run(feat):
        pltpu.async_copy(feat.at[sbuf.at[0]], gbuf.at[0], semg)

        def step(j, carry):
            b = lax.rem(j, 2)
            pltpu.make_async_copy(feat.at[sbuf.at[j]], gbuf.at[b],
                                  semg).wait()

            @pl.when(j >= 1)
            def _():
                pltpu.make_async_copy(gbuf.at[1 - b],
                                      acc.at[dbuf.at[j - 1]], sems).wait()

            pltpu.async_copy(gbuf.at[b], acc.at[dbuf.at[j]], sems, add=True)

            @pl.when(j + 1 < cpt)
            def _():
                pltpu.async_copy(feat.at[sbuf.at[j + 1]],
                                 gbuf.at[1 - b], semg)

            return carry

        lax.fori_loop(0, cpt, step, 0)
        pltpu.make_async_copy(gbuf.at[(cpt - 1) % 2],
                              acc.at[dbuf.at[cpt - 1]], sems).wait()

    @pl.when(c == 0)
    def _():
        _run(feat0)

    @pl.when(c == 1)
    def _():
        _run(feat1)

    plsc.subcore_barrier()

    def _copy_out(row0, nrows):
        done = 0
        for ch in (96, 96, 96, 88, 72):
            take = min(ch, nrows - done)
            if take <= 0:
                break
            pltpu.sync_copy(acc.at[pl.ds(row0 + done, take)],
                            gbuf.at[0, pl.ds(0, take)])
            pltpu.sync_copy(gbuf.at[0, pl.ds(0, take)],
                            out.at[c, pl.ds(row0 + done, take)])
            done += take

    @pl.when(s < 15)
    def _():
        _copy_out(s * 376, 376)

    @pl.when(s == 15)
    def _():
        _copy_out(15 * 376, 360)


def _edge_agg(feat0, feat1, er, n_acc_rows, cpt):
    mesh = plsc.VectorSubcoreMesh(core_axis_name="c", subcore_axis_name="s",
                                  num_cores=NC, num_subcores=NS)
    f32 = jnp.float32
    scratch = [
        pltpu.VMEM_SHARED((n_acc_rows, HC), f32),
        pltpu.VMEM((cpt, CW), jnp.int32),
        pltpu.VMEM((cpt, CW), jnp.int32),
        pltpu.VMEM((2, CW, HC), f32),
        pltpu.SemaphoreType.DMA,
        pltpu.SemaphoreType.DMA,
    ]
    fn = pl.kernel(functools.partial(_agg_body, n_acc_rows, cpt),
                   out_type=jax.ShapeDtypeStruct((NC, N2, HC), f32),
                   mesh=mesh, scratch_types=scratch)
    return fn(feat0, feat1, er)


_BM = 600


def _tca_body(agg_ref, d0d_ref, d1s_ref, w_ref, b_ref, out0_ref, out1_ref):
    blk = agg_ref[...]
    a = jnp.concatenate((blk[0], blk[1]), axis=-1)
    s1 = lax.rsqrt(jnp.maximum(d0d_ref[0, 0], 1.0))
    s2 = lax.rsqrt(jnp.maximum(d1s_ref[0, 0], 1.0))
    h = jnp.dot(a * s1[:, None], w_ref[...],
                preferred_element_type=jnp.float32) + b_ref[...][None, :]
    x = jnp.maximum(h, 0.0) * s2[:, None]
    out0_ref[...] = x[:, :HC]
    out1_ref[...] = x[:, HC:]


def _tca(agg0, d0d, d1s, W1, b1):
    grid = (N2 // _BM,)
    f32 = jnp.float32
    return pl.pallas_call(
        _tca_body,
        grid=grid,
        in_specs=[
            pl.BlockSpec((NC, _BM, HC), lambda i: (0, i, 0)),
            pl.BlockSpec((1, 1, _BM), lambda i: (i, 0, 0)),
            pl.BlockSpec((1, 1, _BM), lambda i: (i, 0, 0)),
            pl.BlockSpec((IN_C, HID), lambda i: (0, 0)),
            pl.BlockSpec((HID,), lambda i: (0,)),
        ],
        out_specs=[
            pl.BlockSpec((_BM, HC), lambda i: (i, 0)),
            pl.BlockSpec((_BM, HC), lambda i: (i, 0)),
        ],
        out_shape=(
            jax.ShapeDtypeStruct((N2, HC), f32),
            jax.ShapeDtypeStruct((N2, HC), f32),
        ),
    )(agg0, d0d, d1s, W1, b1)


def _tcb_body(agg_ref, d1d_ref, w_ref, b_ref, out_ref):
    blk = agg_ref[...]
    a = jnp.concatenate((blk[0], blk[1]), axis=-1)
    s1 = lax.rsqrt(jnp.maximum(d1d_ref[0, 0], 1.0))
    z = jnp.dot(a * s1[:, None], w_ref[...],
                preferred_element_type=jnp.float32) + b_ref[...][None, :]
    m = jnp.max(z, axis=1, keepdims=True)
    lse = jnp.log(jnp.sum(jnp.exp(z - m), axis=1, keepdims=True)) + m
    out_ref[...] = (z - lse)[:, :OUT_C]


def _tcb(agg1, d1d, W2p, b2p):
    grid = (N2 // _BM,)
    return pl.pallas_call(
        _tcb_body,
        grid=grid,
        in_specs=[
            pl.BlockSpec((NC, _BM, HC), lambda i: (0, i, 0)),
            pl.BlockSpec((1, 1, _BM), lambda i: (i, 0, 0)),
            pl.BlockSpec((HID, OUT_P), lambda i: (0, 0)),
            pl.BlockSpec((OUT_P,), lambda i: (0,)),
        ],
        out_specs=pl.BlockSpec((_BM, OUT_C), lambda i: (i, 0)),
        out_shape=jax.ShapeDtypeStruct((N2, OUT_C), jnp.float32),
    )(agg1, d1d, W2p, b2p)


def kernel(n_id, x0, edge_index0, edge_index1, node_type, local_node_idx,
           emb1, W1, b1, W2, b2):
    tbl = jnp.concatenate([x0, emb1], axis=0)
    lidx8 = local_node_idx[:HP_ROWS]
    nt8 = node_type[:HP_ROWS]
    e0r = edge_index0.reshape(2, NS, E0 // NS // CW, CW)
    e1r = edge_index1.reshape(2, NS, E1 // NS // CW, CW)

    hp0, hp1, d0d, d1s, d1d = _sc1(tbl, lidx8, nt8, e0r, e1r)
    d0d = d0d[:N2].reshape(N2 // _BM, 1, _BM)
    d1s = d1s.reshape(N2 // _BM, 1, _BM)
    d1d = d1d.reshape(N2 // _BM, 1, _BM)
    agg0 = _edge_agg(hp0, hp1, e0r, n_acc_rows=N1, cpt=E0 // NS // CW)
    xp0, xp1 = _tca(agg0, d0d, d1s, W1, b1)
    agg1 = _edge_agg(xp0, xp1, e1r, n_acc_rows=N2, cpt=E1 // NS // CW)

    W2p = jnp.pad(W2, ((0, 0), (0, OUT_P - OUT_C)))
    b2p = jnp.pad(b2, (0, OUT_P - OUT_C), constant_values=-1e30)
    return _tcb(agg1, d1d, W2p, b2p)
